# baseline (speedup 1.0000x reference)
import functools

import jax
import jax.numpy as jnp
from jax import lax
from jax.experimental import pallas as pl
from jax.experimental.pallas import tpu as pltpu
from jax.experimental.pallas import tpu_sc as plsc

_B = 16
_LQ = 30
_N = 10000
_LC = 25
_V = 100000
_D = 256
_K = 6

_NW = 32
_QPAD = 512
_NP = 10240
_PER_W = _NP // _NW
_CHUNK_S = 32
_N_CHUNK = _PER_W // _CHUNK_S
_IDS_PER_CHUNK = _CHUNK_S * _LC
_GROW = 80
_NG = _IDS_PER_CHUNK // _GROW
_VB = 4000
_VPAD = 104000
_NEG = -1e30

def _wid():
    return lax.axis_index("s") * 2 + lax.axis_index("c")


@functools.lru_cache(maxsize=None)
def _sc_kernels():
    mesh = plsc.VectorSubcoreMesh(core_axis_name="c", subcore_axis_name="s")
    cp = pltpu.CompilerParams(use_tc_tiling_on_sc=False,
                              needs_layout_passes=False)

    @functools.partial(
        pl.kernel, mesh=mesh, compiler_params=cp,
        out_type=jax.ShapeDtypeStruct((_QPAD, _D), jnp.float32),
        scratch_types=[
            pltpu.VMEM((_QPAD // _NW,), jnp.int32),
            pltpu.VMEM((_QPAD // _NW, _D), jnp.float32),
            pltpu.SemaphoreType.DMA,
        ],
    )
    def _sc_gather_q(e_hbm, idx_hbm, out_hbm, idx_v, rows_v, sem):
        rw = _QPAD // _NW
        base = _wid() * rw
        pltpu.sync_copy(idx_hbm.at[pl.ds(base, rw)], idx_v)
        pltpu.async_copy(e_hbm.at[idx_v], rows_v, sem).wait()
        pltpu.sync_copy(rows_v, out_hbm.at[pl.ds(base, rw)])

    @functools.partial(
        pl.kernel, mesh=mesh, compiler_params=cp,
        out_type=jax.ShapeDtypeStruct((_NP, 16), jnp.float32),
        scratch_types=[
            pltpu.VMEM((_NG, _GROW), jnp.int32),
            pltpu.VMEM((_NG, _GROW), jnp.int32),
            pltpu.VMEM((_PER_W,), jnp.int32),
            pltpu.VMEM((_NG, _GROW), jnp.int32),
            pltpu.VMEM((_NG, _GROW), jnp.int32),
            pltpu.VMEM((_IDS_PER_CHUNK, 16), jnp.float32),
            pltpu.VMEM((_IDS_PER_CHUNK, 16), jnp.float32),
            pltpu.VMEM((_CHUNK_S, 16), jnp.float32),
            pltpu.SemaphoreType.DMA,
            pltpu.SemaphoreType.DMA,
        ],
    )
    def _sc_pool_scores(ptab_hbm, cids_hbm, clen_hbm, tokpat_hbm, sentpat_hbm,
                        out_hbm, idx0, idx1, clen_v, tok_v, sent_v, rows0,
                        rows1, acc_v, sem0, sem1):
        w = _wid()
        pltpu.sync_copy(tokpat_hbm, tok_v)
        pltpu.sync_copy(sentpat_hbm, sent_v)
        pltpu.sync_copy(clen_hbm.at[pl.ds(w * _PER_W, _PER_W)], clen_v)

        def launch(ci, idx_v, rows_v, sem):
            rb = w * (_PER_W * _LC // _GROW) + ci * _NG
            pltpu.sync_copy(cids_hbm.at[pl.ds(rb, _NG)], idx_v)
            for r in range(_NG):
                for cc in range(_GROW // 16):
                    sl = pl.ds(cc * 16, 16)
                    tok = tok_v[r, sl]
                    sent = sent_v[r, sl] + ci * _CHUNK_S
                    cl = plsc.load_gather(clen_v, [sent])
                    idx_v[r, sl] = jnp.where(tok < cl, idx_v[r, sl], _V)
            for r in range(_NG):
                pltpu.async_copy(ptab_hbm.at[idx_v.at[r]],
                                 rows_v.at[pl.ds(r * _GROW, _GROW)], sem)

        def drain(ci, rows_v, sem):
            pltpu.make_async_copy(
                ptab_hbm.at[pl.ds(0, _IDS_PER_CHUNK)], rows_v, sem).wait()
            for s in range(_CHUNK_S):
                acc = rows_v[s * _LC, :]
                for l in range(1, _LC):
                    acc = acc + rows_v[s * _LC + l, :]
                acc_v[s, :] = acc
            sb = w * _PER_W + ci * _CHUNK_S
            pltpu.sync_copy(acc_v, out_hbm.at[pl.ds(sb, _CHUNK_S)])

        launch(0, idx0, rows0, sem0)

        @pl.loop(0, _N_CHUNK, step=2)
        def _(ci):
            launch(ci + 1, idx1, rows1, sem1)
            drain(ci, rows0, sem0)

            @pl.when(ci + 2 < _N_CHUNK)
            def _():
                launch(ci + 2, idx0, rows0, sem0)
            drain(ci + 1, rows1, sem1)

    @functools.partial(
        pl.kernel, mesh=mesh, compiler_params=cp,
        out_type=(jax.ShapeDtypeStruct((256, 80), jnp.int32),
                  jax.ShapeDtypeStruct((256, _LC * 16), jnp.int32)),
        scratch_types=[
            pltpu.VMEM((8,), jnp.int32),
            pltpu.VMEM((8, 80), jnp.int32),
            pltpu.VMEM((8, _LC * 16), jnp.int32),
            pltpu.SemaphoreType.DMA,
        ],
    )
    def _sc_gather_ctx(c2_hbm, cc2_hbm, idx_hbm, o1_hbm, o2_hbm,
                       idx_v, r1, r2, sem):
        base = _wid() * 8
        pltpu.sync_copy(idx_hbm.at[pl.ds(base, 8)], idx_v)
        pltpu.async_copy(c2_hbm.at[idx_v], r1, sem).wait()
        pltpu.sync_copy(r1, o1_hbm.at[pl.ds(base, 8)])
        pltpu.async_copy(cc2_hbm.at[idx_v], r2, sem).wait()
        pltpu.sync_copy(r2, o2_hbm.at[pl.ds(base, 8)])

    return _sc_gather_q, _sc_pool_scores, _sc_gather_ctx


def _tc_ptab_body(e_ref, qrows_ref, w_ref, qlenf_ref, out_ref, qwt_ref):
    i = pl.program_id(0)

    @pl.when(i == 0)
    def _():
        qr = qrows_ref[...].reshape(_B, 32, _D)
        pos = lax.broadcasted_iota(jnp.int32, (_B, 32, _D), 1).astype(jnp.float32)
        qlen3 = qlenf_ref[...].reshape(_B, 1, _D)
        qsum = jnp.sum(jnp.where(pos < qlen3, qr, 0.0), axis=1)
        qemb = qsum / jnp.maximum(qlenf_ref[...], 1.0)
        qwt_ref[...] = lax.dot_general(
            w_ref[...], qemb, (((0,), (1,)), ((), ())),
            preferred_element_type=jnp.float32)

    rows = lax.broadcasted_iota(jnp.int32, (_VB, 16), 0) + i * _VB
    val = lax.dot_general(
        e_ref[...], qwt_ref[...], (((1,), (0,)), ((), ())),
        preferred_element_type=jnp.float32)
    out_ref[...] = jnp.where(rows < _V, val, 0.0)


def _tc_scores_body(cpool_ref, clenf_ref, u_ref, rouge_ref,
                    cs_ref, loss_ref, topk_ref, cl_ref, ctxlen_ref):
    ii = lax.broadcasted_iota(jnp.int32, (_B, _B), 0)
    jj = lax.broadcasted_iota(jnp.int32, (_B, _B), 1)
    eye = (ii == jj).astype(jnp.float32)
    pooled = lax.dot_general(
        eye, cpool_ref[...], (((0,), (1,)), ((), ())),
        preferred_element_type=jnp.float32)[:, :_N]

    lane = lax.broadcasted_iota(jnp.int32, (_B, _N), 1)
    clen_row = clenf_ref[...]
    scores = pooled / jnp.maximum(clen_row, 1.0)
    u = u_ref[...]
    g = -jnp.log(-jnp.log(u + 1e-20) + 1e-20)
    z = scores + g
    rmax = jnp.max(z, axis=1, keepdims=True)
    ex = jnp.exp(z - rmax)
    ssum = jnp.sum(ex, axis=1, keepdims=True)
    cs = (z - rmax) - jnp.log(ssum)
    cs_ref[...] = cs

    rg = rouge_ref[...]
    rmx = jnp.max(rg, axis=1, keepdims=True)
    big = jnp.int32(2**30)
    bidx = jnp.min(jnp.where(rg == rmx, lane, big), axis=1, keepdims=True)
    sel = jnp.sum(jnp.where(lane == bidx, cs, 0.0), axis=1, keepdims=True)
    loss_ref[...] = jnp.broadcast_to(-jnp.mean(sel), (8, 128))

    lane8 = lax.broadcasted_iota(jnp.int32, (_B, 8), 1)
    topk = jnp.zeros((_B, 8), jnp.int32)
    cl8 = jnp.zeros((_B, 8), jnp.int32)
    ctxlen = jnp.zeros((_B, 1), jnp.int32)
    cur = z
    for j in range(_K):
        m = jnp.max(cur, axis=1, keepdims=True)
        ij = jnp.min(jnp.where(cur == m, lane, big), axis=1, keepdims=True)
        hit = lane == ij
        clj = jnp.sum(jnp.where(hit, clen_row, 0.0), axis=1,
                      keepdims=True).astype(jnp.int32)
        topk = jnp.where(lane8 == j, jnp.broadcast_to(ij, (_B, 8)), topk)
        cl8 = jnp.where(lane8 == j, jnp.broadcast_to(clj, (_B, 8)), cl8)
        ctxlen = ctxlen + clj
        cur = jnp.where(hit, _NEG, cur)
    topk_ref[...] = topk
    cl_ref[...] = cl8
    ctxlen_ref[...] = jnp.broadcast_to(ctxlen, (_B, 8))


def _tc_pad_c_body(x_ref, o_ref):
    o_ref[...] = jnp.concatenate(
        [x_ref[...], jnp.zeros((x_ref.shape[0], 80 - _LC * 3), jnp.int32)],
        axis=1)


def _tc_mask_ctx_body(ctx_ref, chars_ref, cl_ref, ctx_o_ref, chars_o_ref):
    lane8 = lax.broadcasted_iota(jnp.int32, (_B, 8), 1)
    e80 = lax.broadcasted_iota(jnp.int32, (_B, 80), 1)
    l80 = e80 // 3
    l400 = lax.broadcasted_iota(jnp.int32, (_B, _LC * 16), 1) // 16
    cl = cl_ref[...]
    for j in range(_K):
        clj = jnp.sum(jnp.where(lane8 == j, cl, 0), axis=1, keepdims=True)
        sl = pl.ds(j * _B, _B)
        m80 = (l80 < clj) & (e80 < _LC * 3)
        ctx_o_ref[sl, :] = jnp.where(m80, ctx_ref[sl, :], 0)
        chars_o_ref[sl, :] = jnp.where(l400 < clj, chars_ref[sl, :], 0)


def kernel(q, q_chars, c, c_chars, c_rouge, qlen, clen, E, W):
    q_ids = jnp.pad(q[:, :, 0].astype(jnp.int32),
                    ((0, 0), (0, 32 - _LQ))).reshape(_QPAD)
    qlenf = jnp.broadcast_to(qlen.astype(jnp.float32)[:, None], (_B, _D))
    cids = jnp.pad(c[:, :, 0].astype(jnp.int32), ((0, _NP - _N), (0, 0)),
                   constant_values=_V).reshape(_NP * _LC // _GROW, _GROW)
    clen_i = jnp.pad(clen.astype(jnp.int32), (0, _NP - _N),
                     constant_values=1)
    clenf_row = clen.astype(jnp.float32).reshape(1, _N)
    u = jax.random.uniform(jax.random.key(42), (_B, _N), dtype=jnp.float32)

    sc_gather_q, sc_pool_scores, sc_gather_ctx = _sc_kernels()

    qrows = sc_gather_q(E, q_ids)

    n_blk = _VPAD // _VB
    ptab = pl.pallas_call(
        _tc_ptab_body,
        grid=(n_blk,),
        in_specs=[
            pl.BlockSpec((_VB, _D), lambda i: (jnp.minimum(i, _V // _VB - 1), 0)),
            pl.BlockSpec((_QPAD, _D), lambda i: (0, 0)),
            pl.BlockSpec((_D, _D), lambda i: (0, 0)),
            pl.BlockSpec((_B, _D), lambda i: (0, 0)),
        ],
        out_specs=pl.BlockSpec((_VB, 16), lambda i: (i, 0)),
        out_shape=jax.ShapeDtypeStruct((_VPAD, 16), jnp.float32),
        scratch_shapes=[pltpu.VMEM((_D, 16), jnp.float32)],
    )(E, qrows, W, qlenf)

    pos = jnp.arange(_IDS_PER_CHUNK, dtype=jnp.int32)
    tokpat = (pos % _LC).reshape(_NG, _GROW)
    sentpat = (pos // _LC).reshape(_NG, _GROW)
    cpool = sc_pool_scores(ptab, cids, clen_i, tokpat, sentpat)

    c_scores, loss88, topk8, cl8, ctxlen8 = pl.pallas_call(
        _tc_scores_body,
        in_specs=[
            pl.BlockSpec((_NP, 16), lambda: (0, 0)),
            pl.BlockSpec((1, _N), lambda: (0, 0)),
            pl.BlockSpec((_B, _N), lambda: (0, 0)),
            pl.BlockSpec((_B, _N), lambda: (0, 0)),
        ],
        out_specs=[
            pl.BlockSpec((_B, _N), lambda: (0, 0)),
            pl.BlockSpec((8, 128), lambda: (0, 0)),
            pl.BlockSpec((_B, 8), lambda: (0, 0)),
            pl.BlockSpec((_B, 8), lambda: (0, 0)),
            pl.BlockSpec((_B, 8), lambda: (0, 0)),
        ],
        out_shape=[
            jax.ShapeDtypeStruct((_B, _N), jnp.float32),
            jax.ShapeDtypeStruct((8, 128), jnp.float32),
            jax.ShapeDtypeStruct((_B, 8), jnp.int32),
            jax.ShapeDtypeStruct((_B, 8), jnp.int32),
            jax.ShapeDtypeStruct((_B, 8), jnp.int32),
        ],
    )(cpool, clenf_row, u, c_rouge)

    ir1_loss = loss88[0, 0]
    topk_idx = topk8[:, :_K]
    ctx_len = ctxlen8[:, 0]

    tk_flat = jnp.pad(topk8[:, :_K].T.reshape(_B * _K), (0, 256 - _B * _K))
    c2 = pl.pallas_call(
        _tc_pad_c_body,
        grid=(5,),
        in_specs=[pl.BlockSpec((_N // 5, _LC * 3), lambda i: (i, 0))],
        out_specs=pl.BlockSpec((_N // 5, 80), lambda i: (i, 0)),
        out_shape=jax.ShapeDtypeStruct((_N, 80), jnp.int32),
    )(c.reshape(_N, _LC * 3).astype(jnp.int32))
    cc2 = c_chars.reshape(_N, _LC * 16).astype(jnp.int32)
    ctx_raw, chars_raw = sc_gather_ctx(c2, cc2, tk_flat)
    ctx_raw = ctx_raw[:_B * _K]
    chars_raw = chars_raw[:_B * _K]

    ctx_m, chars_m = pl.pallas_call(
        _tc_mask_ctx_body,
        in_specs=[
            pl.BlockSpec((_B * _K, 80), lambda: (0, 0)),
            pl.BlockSpec((_B * _K, _LC * 16), lambda: (0, 0)),
            pl.BlockSpec((_B, 8), lambda: (0, 0)),
        ],
        out_specs=[
            pl.BlockSpec((_B * _K, 80), lambda: (0, 0)),
            pl.BlockSpec((_B * _K, _LC * 16), lambda: (0, 0)),
        ],
        out_shape=[
            jax.ShapeDtypeStruct((_B * _K, 80), jnp.int32),
            jax.ShapeDtypeStruct((_B * _K, _LC * 16), jnp.int32),
        ],
    )(ctx_raw, chars_raw, cl8)

    ctx = (ctx_m.reshape(_K, _B, 80)[:, :, :_LC * 3]
           .transpose(1, 0, 2).reshape(_B, _K * _LC, 3))
    ctx_chars = (chars_m.reshape(_K, _B, _LC * 16)
                 .transpose(1, 0, 2).reshape(_B, _K * _LC, 16))
    return (c_scores, ir1_loss, ctx, ctx_chars, ctx_len, topk_idx)


# speedup vs baseline: 2.6940x; 2.6940x over previous
import functools

import jax
import jax.numpy as jnp
from jax import lax
from jax.experimental import pallas as pl
from jax.experimental.pallas import tpu as pltpu
from jax.experimental.pallas import tpu_sc as plsc

_B = 16
_LQ = 30
_N = 10000
_LC = 25
_V = 100000
_D = 256
_K = 6

_NW = 32
_QPAD = 512
_NP = 10240
_PER_W = _NP // _NW
_CHUNK_S = 32
_N_CHUNK = _PER_W // _CHUNK_S
_IDS_PER_CHUNK = _CHUNK_S * _LC
_GROW = 80
_NG = _IDS_PER_CHUNK // _GROW
_VB = 4000
_VPAD = 104000
_NEG = -1e30

def _wid():
    return lax.axis_index("s") * 2 + lax.axis_index("c")


@functools.lru_cache(maxsize=None)
def _sc_kernels():
    mesh = plsc.VectorSubcoreMesh(core_axis_name="c", subcore_axis_name="s")
    cp = pltpu.CompilerParams(use_tc_tiling_on_sc=False,
                              needs_layout_passes=False)

    @functools.partial(
        pl.kernel, mesh=mesh, compiler_params=cp,
        out_type=jax.ShapeDtypeStruct((_QPAD, _D), jnp.float32),
        scratch_types=[
            pltpu.VMEM((_QPAD // _NW,), jnp.int32),
            pltpu.VMEM((_QPAD // _NW, _D), jnp.float32),
            pltpu.SemaphoreType.DMA,
        ],
    )
    def _sc_gather_q(e_hbm, idx_hbm, out_hbm, idx_v, rows_v, sem):
        rw = _QPAD // _NW
        base = _wid() * rw
        pltpu.sync_copy(idx_hbm.at[pl.ds(base, rw)], idx_v)
        pltpu.async_copy(e_hbm.at[idx_v], rows_v, sem).wait()
        pltpu.sync_copy(rows_v, out_hbm.at[pl.ds(base, rw)])

    @functools.partial(
        pl.kernel, mesh=mesh, compiler_params=cp,
        out_type=jax.ShapeDtypeStruct((_NP, 16), jnp.float32),
        scratch_types=[
            pltpu.VMEM((_NG, _GROW), jnp.int32),
            pltpu.VMEM((_NG, _GROW), jnp.int32),
            pltpu.VMEM((_PER_W,), jnp.int32),
            pltpu.VMEM((_NG, _GROW), jnp.int32),
            pltpu.VMEM((_NG, _GROW), jnp.int32),
            pltpu.VMEM((_NG, _GROW), jnp.int32),
            pltpu.VMEM((_IDS_PER_CHUNK, 16), jnp.float32),
            pltpu.VMEM((_IDS_PER_CHUNK, 16), jnp.float32),
            pltpu.VMEM((_CHUNK_S, 16), jnp.float32),
            pltpu.SemaphoreType.DMA,
            pltpu.SemaphoreType.DMA,
        ],
    )
    def _sc_pool_scores(ptab_hbm, cids_hbm, clen_hbm, tokpat_hbm, sentpat_hbm,
                        sntpat_hbm, out_hbm, idx0, idx1, clen_v, tok_v, sent_v,
                        snt_v, rows0, rows1, acc_v, sem0, sem1):
        w = _wid()
        pltpu.sync_copy(tokpat_hbm, tok_v)
        pltpu.sync_copy(sentpat_hbm, sent_v)
        pltpu.sync_copy(sntpat_hbm, snt_v)
        pltpu.sync_copy(clen_hbm.at[pl.ds(w * _PER_W, _PER_W)], clen_v)

        def launch(ci, idx_v, rows_v, sem):
            rb = w * (_PER_W * _LC // _GROW) + ci * _NG
            pltpu.sync_copy(cids_hbm.at[pl.ds(rb, _NG)], idx_v)
            for r in range(_NG):
                for cc in range(_GROW // 16):
                    sl = pl.ds(cc * 16, 16)
                    tok = tok_v[r, sl]
                    sent = sent_v[r, sl] + ci * _CHUNK_S
                    cl = plsc.load_gather(clen_v, [sent])
                    idx_v[r, sl] = jnp.where(tok < cl, idx_v[r, sl],
                                             snt_v[r, sl])
            for r in range(_NG):
                pltpu.async_copy(ptab_hbm.at[idx_v.at[r]],
                                 rows_v.at[pl.ds(r * _GROW, _GROW)], sem)

        def drain(ci, rows_v, sem):
            pltpu.make_async_copy(
                ptab_hbm.at[pl.ds(0, _IDS_PER_CHUNK)], rows_v, sem).wait()
            for s in range(_CHUNK_S):
                acc = rows_v[s * _LC, :]
                for l in range(1, _LC):
                    acc = acc + rows_v[s * _LC + l, :]
                acc_v[s, :] = acc
            sb = w * _PER_W + ci * _CHUNK_S
            pltpu.sync_copy(acc_v, out_hbm.at[pl.ds(sb, _CHUNK_S)])

        launch(0, idx0, rows0, sem0)

        @pl.loop(0, _N_CHUNK, step=2)
        def _(ci):
            launch(ci + 1, idx1, rows1, sem1)
            drain(ci, rows0, sem0)

            @pl.when(ci + 2 < _N_CHUNK)
            def _():
                launch(ci + 2, idx0, rows0, sem0)
            drain(ci + 1, rows1, sem1)

    @functools.partial(
        pl.kernel, mesh=mesh, compiler_params=cp,
        out_type=(jax.ShapeDtypeStruct((256, 80), jnp.int32),
                  jax.ShapeDtypeStruct((256, _LC * 16), jnp.int32)),
        scratch_types=[
            pltpu.VMEM((8,), jnp.int32),
            pltpu.VMEM((8, 80), jnp.int32),
            pltpu.VMEM((8, _LC * 16), jnp.int32),
            pltpu.SemaphoreType.DMA,
        ],
    )
    def _sc_gather_ctx(c2_hbm, cc2_hbm, idx_hbm, o1_hbm, o2_hbm,
                       idx_v, r1, r2, sem):
        base = _wid() * 8
        pltpu.sync_copy(idx_hbm.at[pl.ds(base, 8)], idx_v)
        pltpu.async_copy(c2_hbm.at[idx_v], r1, sem).wait()
        pltpu.sync_copy(r1, o1_hbm.at[pl.ds(base, 8)])
        pltpu.async_copy(cc2_hbm.at[idx_v], r2, sem).wait()
        pltpu.sync_copy(r2, o2_hbm.at[pl.ds(base, 8)])

    return _sc_gather_q, _sc_pool_scores, _sc_gather_ctx


def _tc_ptab_body(e_ref, qrows_ref, w_ref, qlenf_ref, out_ref, qwt_ref):
    i = pl.program_id(0)

    @pl.when(i == 0)
    def _():
        qr = qrows_ref[...].reshape(_B, 32, _D)
        pos = lax.broadcasted_iota(jnp.int32, (_B, 32, _D), 1).astype(jnp.float32)
        qlen3 = qlenf_ref[...].reshape(_B, 1, _D)
        qsum = jnp.sum(jnp.where(pos < qlen3, qr, 0.0), axis=1)
        qemb = qsum / jnp.maximum(qlenf_ref[...], 1.0)
        qwt_ref[...] = lax.dot_general(
            w_ref[...], qemb, (((0,), (1,)), ((), ())),
            preferred_element_type=jnp.float32)

    rows = lax.broadcasted_iota(jnp.int32, (_VB, 16), 0) + i * _VB
    val = lax.dot_general(
        e_ref[...], qwt_ref[...], (((1,), (0,)), ((), ())),
        preferred_element_type=jnp.float32)
    out_ref[...] = jnp.where(rows < _V, val, 0.0)


def _tc_scores_body(cpool_ref, clenf_ref, u_ref, rouge_ref,
                    cs_ref, loss_ref, topk_ref, cl_ref, ctxlen_ref):
    ii = lax.broadcasted_iota(jnp.int32, (_B, _B), 0)
    jj = lax.broadcasted_iota(jnp.int32, (_B, _B), 1)
    eye = (ii == jj).astype(jnp.float32)
    pooled = lax.dot_general(
        eye, cpool_ref[...], (((0,), (1,)), ((), ())),
        preferred_element_type=jnp.float32)[:, :_N]

    lane = lax.broadcasted_iota(jnp.int32, (_B, _N), 1)
    clen_row = clenf_ref[...]
    scores = pooled / jnp.maximum(clen_row, 1.0)
    u = u_ref[...]
    g = -jnp.log(-jnp.log(u + 1e-20) + 1e-20)
    z = scores + g
    rmax = jnp.max(z, axis=1, keepdims=True)
    ex = jnp.exp(z - rmax)
    ssum = jnp.sum(ex, axis=1, keepdims=True)
    cs = (z - rmax) - jnp.log(ssum)
    cs_ref[...] = cs

    rg = rouge_ref[...]
    rmx = jnp.max(rg, axis=1, keepdims=True)
    big = jnp.int32(2**30)
    bidx = jnp.min(jnp.where(rg == rmx, lane, big), axis=1, keepdims=True)
    sel = jnp.sum(jnp.where(lane == bidx, cs, 0.0), axis=1, keepdims=True)
    loss_ref[...] = jnp.broadcast_to(-jnp.mean(sel), (8, 128))

    lane8 = lax.broadcasted_iota(jnp.int32, (_B, 8), 1)
    topk = jnp.zeros((_B, 8), jnp.int32)
    cl8 = jnp.zeros((_B, 8), jnp.int32)
    ctxlen = jnp.zeros((_B, 1), jnp.int32)
    cur = z
    for j in range(_K):
        m = jnp.max(cur, axis=1, keepdims=True)
        ij = jnp.min(jnp.where(cur == m, lane, big), axis=1, keepdims=True)
        hit = lane == ij
        clj = jnp.sum(jnp.where(hit, clen_row, 0.0), axis=1,
                      keepdims=True).astype(jnp.int32)
        topk = jnp.where(lane8 == j, jnp.broadcast_to(ij, (_B, 8)), topk)
        cl8 = jnp.where(lane8 == j, jnp.broadcast_to(clj, (_B, 8)), cl8)
        ctxlen = ctxlen + clj
        cur = jnp.where(hit, _NEG, cur)
    topk_ref[...] = topk
    cl_ref[...] = cl8
    ctxlen_ref[...] = jnp.broadcast_to(ctxlen, (_B, 8))


def _tc_pad_c_body(x_ref, o_ref):
    o_ref[...] = jnp.concatenate(
        [x_ref[...], jnp.zeros((x_ref.shape[0], 80 - _LC * 3), jnp.int32)],
        axis=1)


def _tc_mask_ctx_body(ctx_ref, chars_ref, cl_ref, ctx_o_ref, chars_o_ref):
    lane8 = lax.broadcasted_iota(jnp.int32, (_B, 8), 1)
    e80 = lax.broadcasted_iota(jnp.int32, (_B, 80), 1)
    l80 = e80 // 3
    l400 = lax.broadcasted_iota(jnp.int32, (_B, _LC * 16), 1) // 16
    cl = cl_ref[...]
    for j in range(_K):
        clj = jnp.sum(jnp.where(lane8 == j, cl, 0), axis=1, keepdims=True)
        sl = pl.ds(j * _B, _B)
        m80 = (l80 < clj) & (e80 < _LC * 3)
        ctx_o_ref[sl, :] = jnp.where(m80, ctx_ref[sl, :], 0)
        chars_o_ref[sl, :] = jnp.where(l400 < clj, chars_ref[sl, :], 0)


def kernel(q, q_chars, c, c_chars, c_rouge, qlen, clen, E, W):
    q_ids = jnp.pad(q[:, :, 0].astype(jnp.int32),
                    ((0, 0), (0, 32 - _LQ))).reshape(_QPAD)
    qlenf = jnp.broadcast_to(qlen.astype(jnp.float32)[:, None], (_B, _D))
    pad_rows = _NP - _N
    pad_ids = _V + (jnp.arange(pad_rows * _LC, dtype=jnp.int32)
                    % (_VPAD - _V)).reshape(pad_rows, _LC)
    cids = jnp.concatenate([c[:, :, 0].astype(jnp.int32), pad_ids],
                           axis=0).reshape(_NP * _LC // _GROW, _GROW)
    clen_i = jnp.pad(clen.astype(jnp.int32), (0, _NP - _N),
                     constant_values=1)
    clenf_row = clen.astype(jnp.float32).reshape(1, _N)
    u = jax.random.uniform(jax.random.key(42), (_B, _N), dtype=jnp.float32)

    sc_gather_q, sc_pool_scores, sc_gather_ctx = _sc_kernels()

    qrows = sc_gather_q(E, q_ids)

    n_blk = _VPAD // _VB
    ptab = pl.pallas_call(
        _tc_ptab_body,
        grid=(n_blk,),
        in_specs=[
            pl.BlockSpec((_VB, _D), lambda i: (jnp.minimum(i, _V // _VB - 1), 0)),
            pl.BlockSpec((_QPAD, _D), lambda i: (0, 0)),
            pl.BlockSpec((_D, _D), lambda i: (0, 0)),
            pl.BlockSpec((_B, _D), lambda i: (0, 0)),
        ],
        out_specs=pl.BlockSpec((_VB, 16), lambda i: (i, 0)),
        out_shape=jax.ShapeDtypeStruct((_VPAD, 16), jnp.float32),
        scratch_shapes=[pltpu.VMEM((_D, 16), jnp.float32)],
    )(E, qrows, W, qlenf)

    pos = jnp.arange(_IDS_PER_CHUNK, dtype=jnp.int32)
    tokpat = (pos % _LC).reshape(_NG, _GROW)
    sentpat = (pos // _LC).reshape(_NG, _GROW)
    sntpat = (_V + (pos * ((_VPAD - _V) // _IDS_PER_CHUNK))
              ).reshape(_NG, _GROW)
    cpool = sc_pool_scores(ptab, cids, clen_i, tokpat, sentpat, sntpat)

    c_scores, loss88, topk8, cl8, ctxlen8 = pl.pallas_call(
        _tc_scores_body,
        in_specs=[
            pl.BlockSpec((_NP, 16), lambda: (0, 0)),
            pl.BlockSpec((1, _N), lambda: (0, 0)),
            pl.BlockSpec((_B, _N), lambda: (0, 0)),
            pl.BlockSpec((_B, _N), lambda: (0, 0)),
        ],
        out_specs=[
            pl.BlockSpec((_B, _N), lambda: (0, 0)),
            pl.BlockSpec((8, 128), lambda: (0, 0)),
            pl.BlockSpec((_B, 8), lambda: (0, 0)),
            pl.BlockSpec((_B, 8), lambda: (0, 0)),
            pl.BlockSpec((_B, 8), lambda: (0, 0)),
        ],
        out_shape=[
            jax.ShapeDtypeStruct((_B, _N), jnp.float32),
            jax.ShapeDtypeStruct((8, 128), jnp.float32),
            jax.ShapeDtypeStruct((_B, 8), jnp.int32),
            jax.ShapeDtypeStruct((_B, 8), jnp.int32),
            jax.ShapeDtypeStruct((_B, 8), jnp.int32),
        ],
    )(cpool, clenf_row, u, c_rouge)

    ir1_loss = loss88[0, 0]
    topk_idx = topk8[:, :_K]
    ctx_len = ctxlen8[:, 0]

    tk_flat = jnp.concatenate(
        [topk8[:, :_K].T.reshape(_B * _K),
         jnp.arange(256 - _B * _K, dtype=jnp.int32) * 61])
    c2 = pl.pallas_call(
        _tc_pad_c_body,
        grid=(5,),
        in_specs=[pl.BlockSpec((_N // 5, _LC * 3), lambda i: (i, 0))],
        out_specs=pl.BlockSpec((_N // 5, 80), lambda i: (i, 0)),
        out_shape=jax.ShapeDtypeStruct((_N, 80), jnp.int32),
    )(c.reshape(_N, _LC * 3).astype(jnp.int32))
    cc2 = c_chars.reshape(_N, _LC * 16).astype(jnp.int32)
    ctx_raw, chars_raw = sc_gather_ctx(c2, cc2, tk_flat)
    ctx_raw = ctx_raw[:_B * _K]
    chars_raw = chars_raw[:_B * _K]

    ctx_m, chars_m = pl.pallas_call(
        _tc_mask_ctx_body,
        in_specs=[
            pl.BlockSpec((_B * _K, 80), lambda: (0, 0)),
            pl.BlockSpec((_B * _K, _LC * 16), lambda: (0, 0)),
            pl.BlockSpec((_B, 8), lambda: (0, 0)),
        ],
        out_specs=[
            pl.BlockSpec((_B * _K, 80), lambda: (0, 0)),
            pl.BlockSpec((_B * _K, _LC * 16), lambda: (0, 0)),
        ],
        out_shape=[
            jax.ShapeDtypeStruct((_B * _K, 80), jnp.int32),
            jax.ShapeDtypeStruct((_B * _K, _LC * 16), jnp.int32),
        ],
    )(ctx_raw, chars_raw, cl8)

    ctx = (ctx_m.reshape(_K, _B, 80)[:, :, :_LC * 3]
           .transpose(1, 0, 2).reshape(_B, _K * _LC, 3))
    ctx_chars = (chars_m.reshape(_K, _B, _LC * 16)
                 .transpose(1, 0, 2).reshape(_B, _K * _LC, 16))
    return (c_scores, ir1_loss, ctx, ctx_chars, ctx_len, topk_idx)
